# initial kernel scaffold (unmeasured)
import jax
import jax.numpy as jnp
from jax import lax
from jax.experimental import pallas as pl
from jax.experimental.pallas import tpu as pltpu

N_DEV = 8
SQ = 1024
SKV_LOCAL = 1024
HQ = 8
DH = 128
D = HQ * DH
SCALE = 0.08838834764831843


def kernel(x, Wq, K_ext, V_ext, Wo):
    def body(x_ref, wq_ref, k_ref, v_ref, wo_ref, out_ref,
             comm_o, comm_s, send_sems_o, recv_sems_o,
             send_sems_s, recv_sems_s):
        my = lax.axis_index("i")
        left = lax.rem(my + N_DEV - 1, N_DEV)
        right = lax.rem(my + 1, N_DEV)

        barrier_sem = pltpu.get_barrier_semaphore()
        for nbr in (left, right):
            pl.semaphore_signal(
                barrier_sem, inc=1,
                device_id=(nbr,), device_id_type=pl.DeviceIdType.MESH,
            )
        pl.semaphore_wait(barrier_sem, 2)

        x2 = x_ref[0]
        q2 = jnp.dot(x2, wq_ref[...], preferred_element_type=jnp.float32)
        k3 = k_ref[0]
        v3 = v_ref[0]

        rows = lax.broadcasted_iota(jnp.int32, (SQ, SKV_LOCAL), 0)
        cols = lax.broadcasted_iota(jnp.int32, (SQ, SKV_LOCAL), 1)
        keep = ((rows // 64) % 4) == ((cols // 64) % 4)

        m_parts = []
        l_parts = []
        o_parts = []
        for h in range(HQ):
            qh = q2[:, h * DH:(h + 1) * DH]
            kh = k3[:, h, :]
            vh = v3[:, h, :]
            s = lax.dot_general(
                qh, kh, (((1,), (1,)), ((), ())),
                preferred_element_type=jnp.float32,
            ) * SCALE
            s = jnp.where(keep, s, -1e9)
            mh = jnp.max(s, axis=1, keepdims=True)
            w = jnp.exp(s - mh)
            lh = jnp.sum(w, axis=1, keepdims=True)
            oh = jnp.dot(w, vh, preferred_element_type=jnp.float32)
            m_parts.append(mh)
            l_parts.append(lh)
            o_parts.append(oh[:, None, :])
        acc_m = jnp.concatenate(m_parts, axis=1)
        acc_l = jnp.concatenate(l_parts, axis=1)
        acc_o = jnp.concatenate(o_parts, axis=1)

        comm_o[0] = acc_o
        comm_s[0, :, 0:HQ] = acc_m
        comm_s[0, :, HQ:2 * HQ] = acc_l

        for t in range(N_DEV - 1):
            ss = t % 2
            rs = (t + 1) % 2
            rdma_o = pltpu.make_async_remote_copy(
                src_ref=comm_o.at[ss], dst_ref=comm_o.at[rs],
                send_sem=send_sems_o.at[ss], recv_sem=recv_sems_o.at[rs],
                device_id=(right,), device_id_type=pl.DeviceIdType.MESH,
            )
            rdma_s = pltpu.make_async_remote_copy(
                src_ref=comm_s.at[ss], dst_ref=comm_s.at[rs],
                send_sem=send_sems_s.at[ss], recv_sem=recv_sems_s.at[rs],
                device_id=(right,), device_id_type=pl.DeviceIdType.MESH,
            )
            rdma_o.start()
            rdma_s.start()
            rdma_o.wait()
            rdma_s.wait()

            m2 = comm_s[rs, :, 0:HQ]
            l2 = comm_s[rs, :, HQ:2 * HQ]
            o2 = comm_o[rs]
            new_m = jnp.maximum(acc_m, m2)
            a1 = jnp.exp(acc_m - new_m)
            a2 = jnp.exp(m2 - new_m)
            acc_l = a1 * acc_l + a2 * l2
            acc_o = a1[:, :, None] * acc_o + a2[:, :, None] * o2
            acc_m = new_m

        ctx = acc_o / acc_l[:, :, None]
        ctx2 = ctx.reshape(SQ, D)
        out_ref[0] = jnp.dot(
            ctx2, wo_ref[...], preferred_element_type=jnp.float32
        )

    return pl.pallas_call(
        body,
        out_shape=jax.ShapeDtypeStruct((1, SQ, D), jnp.float32),
        in_specs=[pl.BlockSpec(memory_space=pltpu.VMEM)] * 5,
        out_specs=pl.BlockSpec(memory_space=pltpu.VMEM),
        scratch_shapes=[
            pltpu.VMEM((2, SQ, HQ, DH), jnp.float32),
            pltpu.VMEM((2, SQ, 2 * HQ), jnp.float32),
            pltpu.SemaphoreType.DMA((2,)),
            pltpu.SemaphoreType.DMA((2,)),
            pltpu.SemaphoreType.DMA((2,)),
            pltpu.SemaphoreType.DMA((2,)),
        ],
        compiler_params=pltpu.CompilerParams(collective_id=0),
    )(x, Wq, K_ext, V_ext, Wo)


# baseline (device time: 417603 ns/iter reference)
import jax
import jax.numpy as jnp
from jax import lax
from jax.experimental import pallas as pl
from jax.experimental.pallas import tpu as pltpu

N_DEV = 8
SQ = 1024
SKV_LOCAL = 1024
HQ = 8
DH = 128
D = HQ * DH
SCALE = 0.08838834764831843


def kernel(x, Wq, K_ext, V_ext, Wo):
    def body(x_ref, wq_ref, k_ref, v_ref, wo_ref, out_ref,
             comm_o, comm_s, send_sems_o, recv_sems_o,
             send_sems_s, recv_sems_s):
        my = lax.axis_index("i")
        left = lax.rem(my + N_DEV - 1, N_DEV)
        right = lax.rem(my + 1, N_DEV)

        barrier_sem = pltpu.get_barrier_semaphore()
        for nbr in (left, right):
            pl.semaphore_signal(
                barrier_sem, inc=1,
                device_id=(nbr,), device_id_type=pl.DeviceIdType.MESH,
            )
        pl.semaphore_wait(barrier_sem, 2)

        x2 = x_ref[0]
        q2 = jnp.dot(x2, wq_ref[...], preferred_element_type=jnp.float32)
        k3 = k_ref[0]
        v3 = v_ref[0]

        for g in range(4):
            qg = jnp.concatenate(
                [q2[64 * (g + 4 * c):64 * (g + 4 * c) + 64, :]
                 for c in range(4)], axis=0)
            kg = jnp.concatenate(
                [k3[64 * (g + 4 * c):64 * (g + 4 * c) + 64]
                 for c in range(4)], axis=0)
            vg = jnp.concatenate(
                [v3[64 * (g + 4 * c):64 * (g + 4 * c) + 64]
                 for c in range(4)], axis=0)
            o_parts = []
            m_parts = []
            l_parts = []
            for h in range(HQ):
                qh = qg[:, h * DH:(h + 1) * DH]
                s = lax.dot_general(
                    qh, kg[:, h, :], (((1,), (1,)), ((), ())),
                    preferred_element_type=jnp.float32,
                ) * SCALE
                mh = jnp.max(s, axis=1, keepdims=True)
                w = jnp.exp(s - mh)
                lh = jnp.sum(w, axis=1, keepdims=True)
                oh = jnp.dot(w, vg[:, h, :],
                             preferred_element_type=jnp.float32)
                o_parts.append(oh[:, None, :])
                m_parts.append(mh)
                l_parts.append(lh)
            comm_o[0, pl.ds(256 * g, 256)] = jnp.concatenate(o_parts, axis=1)
            comm_s[0, pl.ds(256 * g, 256), 0:HQ] = jnp.concatenate(
                m_parts, axis=1)
            comm_s[0, pl.ds(256 * g, 256), HQ:2 * HQ] = jnp.concatenate(
                l_parts, axis=1)

        acc_o = comm_o[0]
        acc_m = comm_s[0, :, 0:HQ]
        acc_l = comm_s[0, :, HQ:2 * HQ]

        for t in range(N_DEV - 1):
            ss = t % 2
            rs = (t + 1) % 2
            rdma_o = pltpu.make_async_remote_copy(
                src_ref=comm_o.at[ss], dst_ref=comm_o.at[rs],
                send_sem=send_sems_o.at[ss], recv_sem=recv_sems_o.at[rs],
                device_id=(right,), device_id_type=pl.DeviceIdType.MESH,
            )
            rdma_s = pltpu.make_async_remote_copy(
                src_ref=comm_s.at[ss], dst_ref=comm_s.at[rs],
                send_sem=send_sems_s.at[ss], recv_sem=recv_sems_s.at[rs],
                device_id=(right,), device_id_type=pl.DeviceIdType.MESH,
            )
            rdma_o.start()
            rdma_s.start()
            rdma_o.wait()
            rdma_s.wait()

            m2 = comm_s[rs, :, 0:HQ]
            l2 = comm_s[rs, :, HQ:2 * HQ]
            o2 = comm_o[rs]
            new_m = jnp.maximum(acc_m, m2)
            a1 = jnp.exp(acc_m - new_m)
            a2 = jnp.exp(m2 - new_m)
            acc_l = a1 * acc_l + a2 * l2
            acc_o = a1[:, :, None] * acc_o + a2[:, :, None] * o2
            acc_m = new_m

        ctx = acc_o / acc_l[:, :, None]
        ctx2 = ctx.reshape(SQ, D)
        res = jnp.dot(ctx2, wo_ref[...], preferred_element_type=jnp.float32)
        for g in range(4):
            for c in range(4):
                out_ref[0, pl.ds(64 * (g + 4 * c), 64), :] = (
                    res[256 * g + 64 * c:256 * g + 64 * c + 64, :])

    return pl.pallas_call(
        body,
        out_shape=jax.ShapeDtypeStruct((1, SQ, D), jnp.float32),
        in_specs=[pl.BlockSpec(memory_space=pltpu.VMEM)] * 5,
        out_specs=pl.BlockSpec(memory_space=pltpu.VMEM),
        scratch_shapes=[
            pltpu.VMEM((2, SQ, HQ, DH), jnp.float32),
            pltpu.VMEM((2, SQ, 2 * HQ), jnp.float32),
            pltpu.SemaphoreType.DMA((2,)),
            pltpu.SemaphoreType.DMA((2,)),
            pltpu.SemaphoreType.DMA((2,)),
            pltpu.SemaphoreType.DMA((2,)),
        ],
        compiler_params=pltpu.CompilerParams(
            collective_id=0,
            vmem_limit_bytes=100 * 1024 * 1024,
        ),
    )(x, Wq, K_ext, V_ext, Wo)


# device time: 100034 ns/iter; 4.1746x vs baseline; 4.1746x over previous
import jax
import jax.numpy as jnp
from jax import lax
from jax.experimental import pallas as pl
from jax.experimental.pallas import tpu as pltpu

N_DEV = 8
SQ = 1024
SKV_LOCAL = 1024
HQ = 8
DH = 128
D = HQ * DH
CHUNK = SQ // N_DEV
SCALE = 0.08838834764831843


def kernel(x, Wq, K_ext, V_ext, Wo):
    def body(x_ref, wq_ref, k_ref, v_ref, wo_ref, out_ref,
             part_o, part_s, comm_b, comm_sb, res_ref,
             b_send_o, b_recv_o, b_send_s, b_recv_s,
             ag_send, ag_recv):
        my = lax.axis_index("i")

        barrier_sem = pltpu.get_barrier_semaphore()
        for p in range(1, N_DEV):
            pl.semaphore_signal(
                barrier_sem, inc=1,
                device_id=(lax.rem(my + p, N_DEV),),
                device_id_type=pl.DeviceIdType.MESH,
            )
        pl.semaphore_wait(barrier_sem, N_DEV - 1)

        x2 = x_ref[0]
        q2 = jnp.dot(x2, wq_ref[...], preferred_element_type=jnp.float32)
        k3 = k_ref[0]
        v3 = v_ref[0]

        for g in range(4):
            qg = jnp.concatenate(
                [q2[64 * (g + 4 * c):64 * (g + 4 * c) + 64, :]
                 for c in range(4)], axis=0)
            kg = jnp.concatenate(
                [k3[64 * (g + 4 * c):64 * (g + 4 * c) + 64]
                 for c in range(4)], axis=0)
            vg = jnp.concatenate(
                [v3[64 * (g + 4 * c):64 * (g + 4 * c) + 64]
                 for c in range(4)], axis=0)
            o_parts = []
            m_parts = []
            l_parts = []
            for h in range(HQ):
                qh = qg[:, h * DH:(h + 1) * DH]
                s = lax.dot_general(
                    qh, kg[:, h, :], (((1,), (1,)), ((), ())),
                    preferred_element_type=jnp.float32,
                ) * SCALE
                mh = jnp.max(s, axis=1, keepdims=True)
                w = jnp.exp(s - mh)
                lh = jnp.sum(w, axis=1, keepdims=True)
                oh = jnp.dot(w, vg[:, h, :],
                             preferred_element_type=jnp.float32)
                o_parts.append(oh[:, None, :])
                m_parts.append(mh)
                l_parts.append(lh)
            og = jnp.concatenate(o_parts, axis=1)
            mg = jnp.concatenate(m_parts, axis=1)
            lg = jnp.concatenate(l_parts, axis=1)
            for c in range(4):
                b = g + 4 * c
                part_o[64 * b:64 * b + 64] = og[64 * c:64 * c + 64]
                part_s[64 * b:64 * b + 64, 0:HQ] = mg[64 * c:64 * c + 64]
                part_s[64 * b:64 * b + 64, HQ:2 * HQ] = lg[64 * c:64 * c + 64]

        b_sends = []
        for r in range(N_DEV):
            dst = lax.rem(my + r, N_DEV)
            rd_o = pltpu.make_async_remote_copy(
                src_ref=part_o.at[pl.ds(CHUNK * dst, CHUNK)],
                dst_ref=comm_b.at[my],
                send_sem=b_send_o.at[r], recv_sem=b_recv_o.at[my],
                device_id=(dst,), device_id_type=pl.DeviceIdType.MESH,
            )
            rd_s = pltpu.make_async_remote_copy(
                src_ref=part_s.at[pl.ds(CHUNK * dst, CHUNK)],
                dst_ref=comm_sb.at[my],
                send_sem=b_send_s.at[r], recv_sem=b_recv_s.at[my],
                device_id=(dst,), device_id_type=pl.DeviceIdType.MESH,
            )
            rd_o.start()
            rd_s.start()
            b_sends.append((rd_o, rd_s))

        for s in range(N_DEV):
            pltpu.make_async_remote_copy(
                src_ref=part_o.at[pl.ds(0, CHUNK)], dst_ref=comm_b.at[s],
                send_sem=b_send_o.at[s], recv_sem=b_recv_o.at[s],
                device_id=(my,), device_id_type=pl.DeviceIdType.MESH,
            ).wait_recv()
            pltpu.make_async_remote_copy(
                src_ref=part_s.at[pl.ds(0, CHUNK)], dst_ref=comm_sb.at[s],
                send_sem=b_send_s.at[s], recv_sem=b_recv_s.at[s],
                device_id=(my,), device_id_type=pl.DeviceIdType.MESH,
            ).wait_recv()

        ms = [comm_sb[s, :, 0:HQ] for s in range(N_DEV)]
        M = ms[0]
        for s in range(1, N_DEV):
            M = jnp.maximum(M, ms[s])
        l_sum = jnp.zeros((CHUNK, HQ), jnp.float32)
        o_sum = jnp.zeros((CHUNK, HQ, DH), jnp.float32)
        for s in range(N_DEV):
            a = jnp.exp(ms[s] - M)
            l_sum = l_sum + a * comm_sb[s, :, HQ:2 * HQ]
            o_sum = o_sum + a[:, :, None] * comm_b[s]
        ctx = o_sum / l_sum[:, :, None]
        res_ref[...] = jnp.dot(
            ctx.reshape(CHUNK, D), wo_ref[...],
            preferred_element_type=jnp.float32)

        d_sends = []
        for r in range(N_DEV):
            dst = lax.rem(my + r, N_DEV)
            rd = pltpu.make_async_remote_copy(
                src_ref=res_ref,
                dst_ref=out_ref.at[0, pl.ds(CHUNK * my, CHUNK)],
                send_sem=ag_send.at[r], recv_sem=ag_recv.at[my],
                device_id=(dst,), device_id_type=pl.DeviceIdType.MESH,
            )
            rd.start()
            d_sends.append(rd)

        for s in range(N_DEV):
            pltpu.make_async_remote_copy(
                src_ref=res_ref,
                dst_ref=out_ref.at[0, pl.ds(CHUNK * s, CHUNK)],
                send_sem=ag_send.at[s], recv_sem=ag_recv.at[s],
                device_id=(my,), device_id_type=pl.DeviceIdType.MESH,
            ).wait_recv()

        for rd_o, rd_s in b_sends:
            rd_o.wait_send()
            rd_s.wait_send()
        for rd in d_sends:
            rd.wait_send()

    return pl.pallas_call(
        body,
        out_shape=jax.ShapeDtypeStruct((1, SQ, D), jnp.float32),
        in_specs=[pl.BlockSpec(memory_space=pltpu.VMEM)] * 5,
        out_specs=pl.BlockSpec(memory_space=pltpu.VMEM),
        scratch_shapes=[
            pltpu.VMEM((SQ, HQ, DH), jnp.float32),
            pltpu.VMEM((SQ, 2 * HQ), jnp.float32),
            pltpu.VMEM((N_DEV, CHUNK, HQ, DH), jnp.float32),
            pltpu.VMEM((N_DEV, CHUNK, 2 * HQ), jnp.float32),
            pltpu.VMEM((CHUNK, D), jnp.float32),
            pltpu.SemaphoreType.DMA((N_DEV,)),
            pltpu.SemaphoreType.DMA((N_DEV,)),
            pltpu.SemaphoreType.DMA((N_DEV,)),
            pltpu.SemaphoreType.DMA((N_DEV,)),
            pltpu.SemaphoreType.DMA((N_DEV,)),
            pltpu.SemaphoreType.DMA((N_DEV,)),
        ],
        compiler_params=pltpu.CompilerParams(
            collective_id=0,
            vmem_limit_bytes=100 * 1024 * 1024,
        ),
    )(x, Wq, K_ext, V_ext, Wo)
